# baseline (speedup 1.0000x reference)
import functools

import jax
import jax.numpy as jnp
from jax import lax
from jax.experimental import pallas as pl
from jax.experimental.pallas import tpu as pltpu
from jax.experimental.pallas import tpu_sc as plsc

N = 50000
H = 256
G = 512

NC = 2
NS = 16
NW = NC * NS
SPW = G // NW
HV = H // 16
CH = 128
CHE = CH - 8

IDS_V = 50192
BSTEPS = 16

NEG_INF = float("-inf")


def _sc_pool_body(x_hbm, ids_hbm, mean_hbm, sum_hbm, max_hbm, ids_v, off_v,
                  recip_v, buf_a, buf_b, omean_v, osum_v, omax_v, sem_a,
                  sem_b):
    cid = lax.axis_index("c")
    sid = lax.axis_index("s")
    wid = sid * NC + cid
    seg_base = pl.multiple_of(wid * SPW, SPW)

    pltpu.sync_copy(ids_hbm, ids_v.at[pl.ds(0, N)])
    pad16 = jnp.full((16,), G, jnp.int32)
    for k in range((IDS_V - N) // 16):
        ids_v[pl.ds(N + 16 * k, 16)] = pad16

    lane = lax.broadcasted_iota(jnp.int32, (16,), 0)

    def bsearch(gv):
        def step(_, lohi):
            lo, hi = lohi
            mid = (lo + hi) >> 1
            vals = plsc.load_gather(ids_v, [mid])
            lt = vals < gv
            lo2 = jnp.where(lt, mid + 1, lo)
            hi2 = jnp.where(lt, hi, mid)
            return (lo2, hi2)

        lo0 = jnp.zeros((16,), jnp.int32)
        hi0 = jnp.full((16,), N, jnp.int32)
        lo, _ = lax.fori_loop(0, BSTEPS, step, (lo0, hi0))
        return lo

    a = bsearch(seg_base + lane)
    b = bsearch(seg_base + 1 + lane)
    off_v[pl.ds(0, 16)] = a
    off_v[pl.ds(16, 16)] = b

    cf = jnp.maximum((b - a).astype(jnp.float32), 1.0)
    ci = lax.bitcast_convert_type(cf, jnp.int32)
    rc = lax.bitcast_convert_type(
        jnp.full((16,), 0x7EF311C3, jnp.int32) - ci, jnp.float32)
    for _ in range(4):
        rc = rc * (2.0 - cf * rc)
    recip_v[pl.ds(0, 16)] = rc
    recip_v[pl.ds(16, 16)] = rc

    w_start = off_v[pl.ds(0, 16)][0]
    w_end = off_v[pl.ds(31, 16)][0]
    nch = jnp.maximum(lax.div((w_end - w_start) + (CHE - 1), CHE), 1)

    zero = jnp.zeros((16,), jnp.float32)
    ninf = jnp.full((16,), NEG_INF, jnp.float32)
    init = (tuple(zero for _ in range(HV)), tuple(ninf for _ in range(HV)))

    def base_of(c):
        rb = w_start + c * CHE
        al = jnp.minimum((rb // 8) * 8, N - CH)
        return pl.multiple_of(al, 8), rb

    def start_dma(c, buf, sem):
        base, _ = base_of(c)
        pltpu.make_async_copy(x_hbm.at[pl.ds(base, CH)], buf, sem).start()

    def wait_dma(buf, sem):
        pltpu.make_async_copy(x_hbm.at[pl.ds(0, CH)], buf, sem).wait()

    def compute(c, buf, accs):
        base, r0 = base_of(c)
        shift = r0 - base
        r1 = jnp.minimum(r0 + CHE, w_end)
        is_last = r1 >= w_end

        a_vec = off_v[pl.ds(0, 16)]
        b_vec = off_v[pl.ds(16, 16)]
        cs = plsc.all_reduce_population_count(b_vec < r0)[0]
        s_hi = jnp.where(is_last, SPW,
                         plsc.all_reduce_population_count(a_vec < r1)[0])

        def seg_step(s, accs2):
            os_ = off_v[pl.ds(s, 16)][0]
            oe = off_v[pl.ds(16 + s, 16)][0]
            lo = jnp.maximum(os_, r0)
            hi = jnp.minimum(oe, r1)
            nrows = jnp.maximum(hi - lo, 0)
            rbase = shift + (lo - r0)
            n4 = nrows >> 2

            def acc_row(rr, accs3):
                sums, maxs = accs3
                new_s = []
                new_m = []
                for h in range(HV):
                    v = buf[rr, pl.ds(h * 16, 16)]
                    new_s.append(sums[h] + v)
                    new_m.append(jnp.maximum(maxs[h], v))
                return (tuple(new_s), tuple(new_m))

            def quad_body(q, accs3):
                rq = rbase + 4 * q
                for dr in range(4):
                    accs3 = acc_row(rq + dr, accs3)
                return accs3

            accs2 = lax.fori_loop(0, n4, quad_body, accs2)
            accs2 = lax.fori_loop(4 * n4, nrows,
                                  lambda r, a: acc_row(rbase + r, a), accs2)

            completed = (oe >= r0) & ((oe < r1) | is_last)

            @pl.when(completed)
            def _flush():
                sums, maxs = accs2
                recip = recip_v[pl.ds(s, 16)][0]
                for h in range(HV):
                    omean_v[s, pl.ds(h * 16, 16)] = sums[h] * recip
                    osum_v[s, pl.ds(h * 16, 16)] = sums[h]
                    omax_v[s, pl.ds(h * 16, 16)] = maxs[h]

            cvec = jnp.full((16,), completed.astype(jnp.int32)) == 1
            sums, maxs = accs2
            new_s = tuple(jnp.where(cvec, zero, sv) for sv in sums)
            new_m = tuple(jnp.where(cvec, ninf, mv) for mv in maxs)
            return (new_s, new_m)

        return lax.fori_loop(cs, jnp.maximum(s_hi, cs), seg_step, accs)

    start_dma(0, buf_a, sem_a)

    def pair_body(p, accs):
        c0 = 2 * p
        has_b = c0 + 1 < nch
        wait_dma(buf_a, sem_a)

        @pl.when(has_b)
        def _next_b():
            start_dma(c0 + 1, buf_b, sem_b)

        accs = compute(c0, buf_a, accs)

        @pl.when(has_b)
        def _wait_b():
            wait_dma(buf_b, sem_b)

        @pl.when(c0 + 2 < nch)
        def _next_a():
            start_dma(c0 + 2, buf_a, sem_a)

        return compute(c0 + 1, buf_b, accs)

    npairs = lax.div(nch + 1, 2)
    lax.fori_loop(0, npairs, pair_body, init)

    out_ds = pl.ds(seg_base, SPW)
    pltpu.sync_copy(omean_v, mean_hbm.at[out_ds])
    pltpu.sync_copy(osum_v, sum_hbm.at[out_ds])
    pltpu.sync_copy(omax_v, max_hbm.at[out_ds])


def _finalize_body(mean_ref, sum_ref, max_ref, w_ref, b_ref, g_ref, be_ref,
                   out_ref):
    mean = mean_ref[...]
    sums = sum_ref[...]
    maxs = max_ref[...]

    maxf = jnp.where(maxs == jnp.float32(NEG_INF), 0.0, maxs)

    w0 = w_ref[0:H, :]
    w1 = w_ref[H:2 * H, :]
    w2 = w_ref[2 * H:3 * H, :]
    y = (jnp.dot(mean, w0, preferred_element_type=jnp.float32)
         + jnp.dot(maxf, w1, preferred_element_type=jnp.float32)
         + jnp.dot(sums, w2, preferred_element_type=jnp.float32)
         + b_ref[...])

    mu = jnp.mean(y, axis=1, keepdims=True)
    var = jnp.mean((y - mu) ** 2, axis=1, keepdims=True)
    y = (y - mu) / jnp.sqrt(var + 1e-5) * g_ref[...] + be_ref[...]
    out_ref[...] = 0.5 * y * (1.0 + lax.erf(y / jnp.sqrt(2.0).astype(y.dtype)))


_sc_pool = functools.partial(
    pl.kernel,
    out_type=[
        jax.ShapeDtypeStruct((G, H), jnp.float32),
        jax.ShapeDtypeStruct((G, H), jnp.float32),
        jax.ShapeDtypeStruct((G, H), jnp.float32),
    ],
    mesh=plsc.VectorSubcoreMesh(core_axis_name="c", subcore_axis_name="s"),
    compiler_params=pltpu.CompilerParams(needs_layout_passes=False),
    scratch_types=[
        pltpu.VMEM((IDS_V,), jnp.int32),
        pltpu.VMEM((48,), jnp.int32),
        pltpu.VMEM((32,), jnp.float32),
        pltpu.VMEM((CH, H), jnp.float32),
        pltpu.VMEM((CH, H), jnp.float32),
        pltpu.VMEM((SPW, H), jnp.float32),
        pltpu.VMEM((SPW, H), jnp.float32),
        pltpu.VMEM((SPW, H), jnp.float32),
        pltpu.SemaphoreType.DMA,
        pltpu.SemaphoreType.DMA,
    ],
)(_sc_pool_body)


@jax.jit
def kernel(x, batch, W, b, gamma, beta):
    batch = batch.astype(jnp.int32)
    mean, sums, maxs = _sc_pool(x, batch)

    out = pl.pallas_call(
        _finalize_body,
        out_shape=jax.ShapeDtypeStruct((G, H), jnp.float32),
    )(mean, sums, maxs, W, b.reshape(1, H), gamma.reshape(1, H),
      beta.reshape(1, H))
    return out
